# baseline (speedup 1.0000x reference)
import jax
import jax.numpy as jnp
from jax import lax
from jax.experimental import pallas as pl
from jax.experimental.pallas import tpu as pltpu
from jax.experimental.pallas import tpu_sc as plsc

N = 10000
NP = 10240
G = 512
E = 640000
EL = E + N
EP = 655360
NW = 32
EPW = EP // NW
RPT = NP // 16

_F32 = jnp.float32
_I32 = jnp.int32


def _make_edge_kernel(H, Cp, K, KB):
    HCp = H * Cp
    W = HCp + 16
    nblk = HCp // 16
    bph = Cp // 16
    NSUB = K // KB
    NCH = EPW // K
    RPW = EPW // KB

    def body(srcf, dstf, hrow, btab, out, sbuf0, dbuf0, rows0, brow0,
             sbuf1, dbuf1, rows1, brow1, acc, gsem, bsem, ssem):
        c = lax.axis_index("c")
        s = lax.axis_index("s")
        wid = c * 16 + s
        bufs = [(sbuf0, dbuf0, rows0, brow0), (sbuf1, dbuf1, rows1, brow1)]

        zv = jnp.zeros((16,), _F32)

        def _zb(r, carry):
            for b in range(W // 16):
                rows0[r, pl.ds(b * 16, 16)] = zv
            return carry

        lax.fori_loop(0, K, _zb, 0)
        r0 = pl.multiple_of(s * RPT, RPT)

        def _za(k, carry):
            pltpu.sync_copy(rows0,
                            acc.at[pl.ds(pl.multiple_of(r0 + k * K, K), K)])
            return carry

        lax.fori_loop(0, RPT // K, _za, 0)
        if RPT % K:
            pltpu.sync_copy(
                rows0.at[pl.ds(0, RPT % K)],
                acc.at[pl.ds(pl.multiple_of(r0 + RPT - RPT % K, 8), RPT % K)])
        plsc.subcore_barrier()

        lane = lax.iota(_I32, 16)
        kon = [(lane == hd).astype(_F32) for hd in range(H)]

        def _idx_load(ch, p):
            sb, db, _, _ = bufs[p]
            rowbase = wid * RPW + ch * NSUB
            pltpu.sync_copy(srcf.at[pl.ds(rowbase, NSUB)], sb)
            pltpu.sync_copy(dstf.at[pl.ds(rowbase, NSUB)], db)

        def _gather_start(p):
            sb, db, rows, brow = bufs[p]
            for j in range(NSUB):
                pltpu.async_copy(hrow.at[sb.at[j]],
                                 rows.at[pl.ds(j * KB, KB)], gsem)
                pltpu.async_copy(btab.at[db.at[j]],
                                 brow.at[pl.ds(j * KB, KB)], bsem)

        def _gather_wait(p):
            sb, db, rows, brow = bufs[p]
            for j in range(NSUB):
                pltpu.make_async_copy(hrow.at[sb.at[j]],
                                      rows.at[pl.ds(j * KB, KB)], gsem).wait()
                pltpu.make_async_copy(btab.at[db.at[j]],
                                      brow.at[pl.ds(j * KB, KB)], bsem).wait()

        def _scatter_start(p):
            _, db, rows, _ = bufs[p]
            for j in range(NSUB):
                pltpu.async_copy(rows.at[pl.ds(j * KB, KB)],
                                 acc.at[db.at[j]], ssem, add=True)

        def _scatter_wait(p):
            _, db, rows, _ = bufs[p]
            for j in range(NSUB):
                pltpu.make_async_copy(rows.at[pl.ds(j * KB, KB)],
                                      acc.at[db.at[j]], ssem).wait()

        def _compute(p):
            _, _, rows, brow = bufs[p]

            @plsc.parallel_loop(0, K // 16, 1, unroll=4)
            def _grp(g):
                    e0 = g * 16
                    ev = e0 + lane
                    wvecs = []
                    for hd in range(H):
                        a = plsc.load_gather(
                            rows, [ev, jnp.full((16,), HCp + hd, _I32)])
                        b = plsc.load_gather(
                            brow, [ev, jnp.full((16,), hd, _I32)])
                        z = a + b
                        wvecs.append(jnp.exp(jnp.maximum(z, 0.2 * z)))
                    for l in range(16):
                        e = e0 + l
                        ws = [wv[l] for wv in wvecs]
                        for blk in range(nblk):
                            rows[e, pl.ds(blk * 16, 16)] = (
                                rows[e, pl.ds(blk * 16, 16)] * ws[blk // bph])
                    for hd in range(H):
                        plsc.store_scatter(
                            rows, [ev, jnp.full((16,), HCp + hd, _I32)],
                            wvecs[hd])

        _idx_load(0, 0)
        _gather_start(0)

        def _pair(t, carry):
            @pl.when(t > 0)
            def _():
                _scatter_wait(1)
            _idx_load(2 * t + 1, 1)
            _gather_start(1)
            _gather_wait(0)
            _compute(0)
            _scatter_start(0)

            _scatter_wait(0)
            ch2 = jnp.minimum(2 * t + 2, NCH - 1)
            _idx_load(ch2, 0)
            _gather_start(0)
            _gather_wait(1)
            _compute(1)
            _scatter_start(1)
            return carry

        lax.fori_loop(0, NCH // 2, _pair, 0)
        _scatter_wait(1)
        _gather_wait(0)
        plsc.subcore_barrier()
        pltpu.sync_copy(acc.at[pl.ds(r0, RPT)], out.at[c, pl.ds(r0, RPT)])

    mesh = plsc.VectorSubcoreMesh(core_axis_name="c", subcore_axis_name="s",
                                  num_cores=2, num_subcores=16)
    return pl.kernel(
        body,
        out_type=jax.ShapeDtypeStruct((2, NP, W), _F32),
        mesh=mesh,
        compiler_params=pltpu.CompilerParams(needs_layout_passes=False,
                                             use_tc_tiling_on_sc=False),
        scratch_types=[
            pltpu.VMEM((NSUB, KB), _I32),
            pltpu.VMEM((NSUB, KB), _I32),
            pltpu.VMEM((K, W), _F32),
            pltpu.VMEM((K, 16), _F32),
            pltpu.VMEM((NSUB, KB), _I32),
            pltpu.VMEM((NSUB, KB), _I32),
            pltpu.VMEM((K, W), _F32),
            pltpu.VMEM((K, 16), _F32),
            pltpu.VMEM_SHARED((NP, W), _F32),
            pltpu.SemaphoreType.DMA,
            pltpu.SemaphoreType.DMA,
            pltpu.SemaphoreType.DMA,
        ],
    )


BR = 2048
NB = NP // BR


def _tables_from_h(h, a_src, a_dst, Hn, Cn, Cpn, hrow_ref, b_ref):
    HCpn = Hn * Cpn
    hrow_ref[...] = jnp.zeros((BR, HCpn + 16), _F32)
    b_ref[...] = jnp.zeros((BR, 16), _F32)
    acols, bcols = [], []
    for hd in range(Hn):
        blk = h[:, hd * Cn:(hd + 1) * Cn]
        hrow_ref[:, hd * Cpn:hd * Cpn + Cn] = blk
        acols.append(jnp.sum(blk * a_src[0, hd, :][None, :], axis=1,
                             keepdims=True))
        bcols.append(jnp.sum(blk * a_dst[0, hd, :][None, :], axis=1,
                             keepdims=True))
    hrow_ref[:, HCpn:HCpn + Hn] = jnp.concatenate(acols, axis=1)
    b_ref[:, 0:Hn] = jnp.concatenate(bcols, axis=1)


def _full(shape):
    return pl.BlockSpec(shape, lambda i: tuple(0 for _ in shape))


def _rows(width):
    return pl.BlockSpec((BR, width), lambda i: (i, 0))


def _tc_prep(xp, W1, a_src1, a_dst1):
    def body(x_ref, w_ref, as_ref, ad_ref, hrow_ref, b_ref):
        h = jnp.dot(x_ref[...], w_ref[...], preferred_element_type=_F32)
        _tables_from_h(h, as_ref[...], ad_ref[...], 3, 45, 48,
                       hrow_ref, b_ref)

    return pl.pallas_call(
        body,
        grid=(NB,),
        in_specs=[_rows(9), _full((9, 135)), _full((1, 3, 45)),
                  _full((1, 3, 45))],
        out_specs=[_rows(160), _rows(16)],
        out_shape=[
            jax.ShapeDtypeStruct((NP, 160), _F32),
            jax.ShapeDtypeStruct((NP, 16), _F32),
        ],
    )(xp, W1, a_src1, a_dst1)


def _tc_mid(acc, b, Wn, a_srcn, a_dstn, Hl, Cl, Cpl, Hn, Cn, Cpn):
    Wl = Hl * Cpl + 16
    HCn = Hn * Cn

    def body(acc_ref, b_ref, wn_ref, as_ref, ad_ref, hrow_ref, bt_ref):
        sa = acc_ref[0] + acc_ref[1]
        ycols = []
        for hd in range(Hl):
            num = sa[:, hd * Cpl:hd * Cpl + Cl]
            den = sa[:, Hl * Cpl + hd][:, None] + 1e-16
            ycols.append(num / den)
        y = jnp.concatenate(ycols, axis=1) + b_ref[...][None, :]
        y = jnp.maximum(y, 0.0)
        h = jnp.dot(y, wn_ref[...], preferred_element_type=_F32)
        _tables_from_h(h, as_ref[...], ad_ref[...], Hn, Cn, Cpn,
                       hrow_ref, bt_ref)

    return pl.pallas_call(
        body,
        grid=(NB,),
        in_specs=[pl.BlockSpec((2, BR, Wl), lambda i: (0, i, 0)),
                  _full((Hl * Cl,)), _full((Hl * Cl, HCn)),
                  _full((1, Hn, Cn)), _full((1, Hn, Cn))],
        out_specs=[_rows(Hn * Cpn + 16), _rows(16)],
        out_shape=[
            jax.ShapeDtypeStruct((NP, Hn * Cpn + 16), _F32),
            jax.ShapeDtypeStruct((NP, 16), _F32),
        ],
    )(acc, b, Wn, a_srcn, a_dstn)


def _tc_pool(acc, b3, batch2d):
    NEG = float("-inf")

    def body(acc_ref, b_ref, batch_ref, out_ref, h3_ref):
        sa = acc_ref[0] + acc_ref[1]
        h3_ref[...] = (sa[:, 0:4] / (sa[:, 16][:, None] + 1e-16)
                       + b_ref[...][None, :])
        for gb in range(G // 128):
            gv = gb * 128 + lax.broadcasted_iota(_I32, (1, 128), 1)

            def _nloop(k, accs):
                bb = batch_ref[pl.ds(k * 32, 32), 0:1]
                cmp = bb == gv
                hsl = h3_ref[pl.ds(k * 32, 32), :]
                return tuple(
                    jnp.maximum(accs[cl],
                                jnp.where(cmp, hsl[:, cl:cl + 1], NEG))
                    for cl in range(4))

            init = tuple(jnp.full((32, 128), NEG) for _ in range(4))
            accs = lax.fori_loop(0, NP // 32, _nloop, init)
            pooled = [jnp.max(a, axis=0, keepdims=True) for a in accs]
            m = jnp.maximum(jnp.maximum(pooled[0], pooled[1]),
                            jnp.maximum(pooled[2], pooled[3]))
            zs = [p - m for p in pooled]
            lse = jnp.log(sum(jnp.exp(z) for z in zs))
            for cl in range(4):
                out_ref[cl:cl + 1, pl.ds(gb * 128, 128)] = zs[cl] - lse

    return pl.pallas_call(
        body,
        out_shape=jax.ShapeDtypeStruct((4, G), _F32),
        scratch_shapes=[pltpu.VMEM((NP, 4), _F32)],
    )(acc, b3, batch2d)


def kernel(x, edge_index, batch, W1, a_src1, a_dst1, b1, W2, a_src2, a_dst2,
           b2, W3, a_src3, a_dst3, b3):
    loop = jnp.arange(N, dtype=_I32)
    padv = jnp.full((EP - EL,), N, _I32)
    srcf = jnp.concatenate([edge_index[0].astype(_I32), loop, padv])
    dstf = jnp.concatenate([edge_index[1].astype(_I32), loop, padv])
    batch2d = jnp.concatenate(
        [batch.astype(_I32), jnp.full((NP - N,), 1 << 30, _I32)])[:, None]
    xp = jnp.concatenate([x, jnp.zeros((NP - N, x.shape[1]), _F32)])

    src80 = srcf.reshape(EP // 80, 80)
    dst80 = dstf.reshape(EP // 80, 80)
    src128 = srcf.reshape(EP // 128, 128)
    dst128 = dstf.reshape(EP // 128, 128)

    hrow1, B1 = _tc_prep(xp, W1, a_src1, a_dst1)
    acc1 = _make_edge_kernel(3, 48, 80, 80)(src80, dst80, hrow1, B1)
    hrow2, B2 = _tc_mid(acc1, b1, W2, a_src2, a_dst2, 3, 45, 48, 3, 18, 32)
    acc2 = _make_edge_kernel(3, 32, 160, 80)(src80, dst80, hrow2, B2)
    hrow3, B3 = _tc_mid(acc2, b2, W3, a_src3, a_dst3, 3, 18, 32, 1, 4, 16)
    acc3 = _make_edge_kernel(1, 16, 512, 128)(src128, dst128, hrow3, B3)
    outT = _tc_pool(acc3, b3, batch2d)
    return outT.T


# speedup vs baseline: 1.0362x; 1.0362x over previous
import jax
import jax.numpy as jnp
from jax import lax
from jax.experimental import pallas as pl
from jax.experimental.pallas import tpu as pltpu
from jax.experimental.pallas import tpu_sc as plsc

N = 10000
NP = 10240
G = 512
E = 640000
EL = E + N
EP = 655360
NW = 32
EPW = EP // NW
RPT = NP // 16

_F32 = jnp.float32
_I32 = jnp.int32


def _make_edge_kernel(H, Cp, K, KB):
    HCp = H * Cp
    W = HCp + 16
    nblk = HCp // 16
    bph = Cp // 16
    NSUB = K // KB
    NCH = EPW // K
    RPW = EPW // KB

    def body(srcf, dstf, hrow, btab, out, sbuf0, dbuf0, rows0, brow0,
             sbuf1, dbuf1, rows1, brow1, acc, gsem, bsem, ssem):
        c = lax.axis_index("c")
        s = lax.axis_index("s")
        wid = c * 16 + s
        bufs = [(sbuf0, dbuf0, rows0, brow0), (sbuf1, dbuf1, rows1, brow1)]

        zv = jnp.zeros((16,), _F32)

        def _zb(r, carry):
            for b in range(W // 16):
                rows0[r, pl.ds(b * 16, 16)] = zv
            return carry

        lax.fori_loop(0, K, _zb, 0)
        r0 = pl.multiple_of(s * RPT, RPT)

        def _za(k, carry):
            pltpu.sync_copy(rows0,
                            acc.at[pl.ds(pl.multiple_of(r0 + k * K, K), K)])
            return carry

        lax.fori_loop(0, RPT // K, _za, 0)
        if RPT % K:
            pltpu.sync_copy(
                rows0.at[pl.ds(0, RPT % K)],
                acc.at[pl.ds(pl.multiple_of(r0 + RPT - RPT % K, 8), RPT % K)])
        plsc.subcore_barrier()

        lane = lax.iota(_I32, 16)
        kon = [(lane == hd).astype(_F32) for hd in range(H)]

        def _idx_load(ch, p):
            sb, db, _, _ = bufs[p]
            rowbase = wid * RPW + ch * NSUB
            pltpu.sync_copy(srcf.at[pl.ds(rowbase, NSUB)], sb)
            pltpu.sync_copy(dstf.at[pl.ds(rowbase, NSUB)], db)

        def _gather_start(p):
            sb, db, rows, brow = bufs[p]
            for j in range(NSUB):
                pltpu.async_copy(hrow.at[sb.at[j]],
                                 rows.at[pl.ds(j * KB, KB)], gsem)
                pltpu.async_copy(btab.at[db.at[j]],
                                 brow.at[pl.ds(j * KB, KB)], bsem)

        def _gather_wait(p):
            sb, db, rows, brow = bufs[p]
            for j in range(NSUB):
                pltpu.make_async_copy(hrow.at[sb.at[j]],
                                      rows.at[pl.ds(j * KB, KB)], gsem).wait()
                pltpu.make_async_copy(btab.at[db.at[j]],
                                      brow.at[pl.ds(j * KB, KB)], bsem).wait()

        def _scatter_start(p):
            _, db, rows, _ = bufs[p]
            for j in range(NSUB):
                pltpu.async_copy(rows.at[pl.ds(j * KB, KB)],
                                 acc.at[db.at[j]], ssem, add=True)

        def _scatter_wait(p):
            _, db, rows, _ = bufs[p]
            for j in range(NSUB):
                pltpu.make_async_copy(rows.at[pl.ds(j * KB, KB)],
                                      acc.at[db.at[j]], ssem).wait()

        def _compute(p):
            _, _, rows, brow = bufs[p]

            @plsc.parallel_loop(0, K // 16, 1, unroll=2)
            def _grp(g):
                    e0 = g * 16
                    ev = e0 + lane
                    wvecs = []
                    for hd in range(H):
                        a = plsc.load_gather(
                            rows, [ev, jnp.full((16,), HCp + hd, _I32)])
                        b = plsc.load_gather(
                            brow, [ev, jnp.full((16,), hd, _I32)])
                        z = a + b
                        wvecs.append(jnp.exp(jnp.maximum(z, 0.2 * z)))
                    for l in range(16):
                        e = e0 + l
                        ws = [wv[l] for wv in wvecs]
                        for blk in range(nblk):
                            rows[e, pl.ds(blk * 16, 16)] = (
                                rows[e, pl.ds(blk * 16, 16)] * ws[blk // bph])
                    for hd in range(H):
                        plsc.store_scatter(
                            rows, [ev, jnp.full((16,), HCp + hd, _I32)],
                            wvecs[hd])

        _idx_load(0, 0)
        _gather_start(0)

        def _pair(t, carry):
            @pl.when(t > 0)
            def _():
                _scatter_wait(1)
            _idx_load(2 * t + 1, 1)
            _gather_start(1)
            _gather_wait(0)
            _compute(0)
            _scatter_start(0)

            _scatter_wait(0)
            ch2 = jnp.minimum(2 * t + 2, NCH - 1)
            _idx_load(ch2, 0)
            _gather_start(0)
            _gather_wait(1)
            _compute(1)
            _scatter_start(1)
            return carry

        lax.fori_loop(0, NCH // 2, _pair, 0)
        _scatter_wait(1)
        _gather_wait(0)
        plsc.subcore_barrier()
        pltpu.sync_copy(acc.at[pl.ds(r0, RPT)], out.at[c, pl.ds(r0, RPT)])

    mesh = plsc.VectorSubcoreMesh(core_axis_name="c", subcore_axis_name="s",
                                  num_cores=2, num_subcores=16)
    return pl.kernel(
        body,
        out_type=jax.ShapeDtypeStruct((2, NP, W), _F32),
        mesh=mesh,
        compiler_params=pltpu.CompilerParams(needs_layout_passes=False,
                                             use_tc_tiling_on_sc=False),
        scratch_types=[
            pltpu.VMEM((NSUB, KB), _I32),
            pltpu.VMEM((NSUB, KB), _I32),
            pltpu.VMEM((K, W), _F32),
            pltpu.VMEM((K, 16), _F32),
            pltpu.VMEM((NSUB, KB), _I32),
            pltpu.VMEM((NSUB, KB), _I32),
            pltpu.VMEM((K, W), _F32),
            pltpu.VMEM((K, 16), _F32),
            pltpu.VMEM_SHARED((NP, W), _F32),
            pltpu.SemaphoreType.DMA,
            pltpu.SemaphoreType.DMA,
            pltpu.SemaphoreType.DMA,
        ],
    )


BR = 2048
NB = NP // BR


def _tables_from_h(h, a_src, a_dst, Hn, Cn, Cpn, hrow_ref, b_ref):
    HCpn = Hn * Cpn
    hrow_ref[...] = jnp.zeros((BR, HCpn + 16), _F32)
    b_ref[...] = jnp.zeros((BR, 16), _F32)
    acols, bcols = [], []
    for hd in range(Hn):
        blk = h[:, hd * Cn:(hd + 1) * Cn]
        hrow_ref[:, hd * Cpn:hd * Cpn + Cn] = blk
        acols.append(jnp.sum(blk * a_src[0, hd, :][None, :], axis=1,
                             keepdims=True))
        bcols.append(jnp.sum(blk * a_dst[0, hd, :][None, :], axis=1,
                             keepdims=True))
    hrow_ref[:, HCpn:HCpn + Hn] = jnp.concatenate(acols, axis=1)
    b_ref[:, 0:Hn] = jnp.concatenate(bcols, axis=1)


def _full(shape):
    return pl.BlockSpec(shape, lambda i: tuple(0 for _ in shape))


def _rows(width):
    return pl.BlockSpec((BR, width), lambda i: (i, 0))


def _tc_prep(xp, W1, a_src1, a_dst1):
    def body(x_ref, w_ref, as_ref, ad_ref, hrow_ref, b_ref):
        h = jnp.dot(x_ref[...], w_ref[...], preferred_element_type=_F32)
        _tables_from_h(h, as_ref[...], ad_ref[...], 3, 45, 48,
                       hrow_ref, b_ref)

    return pl.pallas_call(
        body,
        grid=(NB,),
        in_specs=[_rows(9), _full((9, 135)), _full((1, 3, 45)),
                  _full((1, 3, 45))],
        out_specs=[_rows(160), _rows(16)],
        out_shape=[
            jax.ShapeDtypeStruct((NP, 160), _F32),
            jax.ShapeDtypeStruct((NP, 16), _F32),
        ],
    )(xp, W1, a_src1, a_dst1)


def _tc_mid(acc, b, Wn, a_srcn, a_dstn, Hl, Cl, Cpl, Hn, Cn, Cpn):
    Wl = Hl * Cpl + 16
    HCn = Hn * Cn

    def body(acc_ref, b_ref, wn_ref, as_ref, ad_ref, hrow_ref, bt_ref):
        sa = acc_ref[0] + acc_ref[1]
        ycols = []
        for hd in range(Hl):
            num = sa[:, hd * Cpl:hd * Cpl + Cl]
            den = sa[:, Hl * Cpl + hd][:, None] + 1e-16
            ycols.append(num / den)
        y = jnp.concatenate(ycols, axis=1) + b_ref[...][None, :]
        y = jnp.maximum(y, 0.0)
        h = jnp.dot(y, wn_ref[...], preferred_element_type=_F32)
        _tables_from_h(h, as_ref[...], ad_ref[...], Hn, Cn, Cpn,
                       hrow_ref, bt_ref)

    return pl.pallas_call(
        body,
        grid=(NB,),
        in_specs=[pl.BlockSpec((2, BR, Wl), lambda i: (0, i, 0)),
                  _full((Hl * Cl,)), _full((Hl * Cl, HCn)),
                  _full((1, Hn, Cn)), _full((1, Hn, Cn))],
        out_specs=[_rows(Hn * Cpn + 16), _rows(16)],
        out_shape=[
            jax.ShapeDtypeStruct((NP, Hn * Cpn + 16), _F32),
            jax.ShapeDtypeStruct((NP, 16), _F32),
        ],
    )(acc, b, Wn, a_srcn, a_dstn)


def _tc_pool(acc, b3, batch2d):
    NEG = float("-inf")

    def body(acc_ref, b_ref, batch_ref, out_ref, h3_ref):
        sa = acc_ref[0] + acc_ref[1]
        h3_ref[...] = (sa[:, 0:4] / (sa[:, 16][:, None] + 1e-16)
                       + b_ref[...][None, :])
        for gb in range(G // 128):
            gv = gb * 128 + lax.broadcasted_iota(_I32, (1, 128), 1)

            def _nloop(k, accs):
                bb = batch_ref[pl.ds(k * 32, 32), 0:1]
                cmp = bb == gv
                hsl = h3_ref[pl.ds(k * 32, 32), :]
                return tuple(
                    jnp.maximum(accs[cl],
                                jnp.where(cmp, hsl[:, cl:cl + 1], NEG))
                    for cl in range(4))

            init = tuple(jnp.full((32, 128), NEG) for _ in range(4))
            accs = lax.fori_loop(0, NP // 32, _nloop, init)
            pooled = [jnp.max(a, axis=0, keepdims=True) for a in accs]
            m = jnp.maximum(jnp.maximum(pooled[0], pooled[1]),
                            jnp.maximum(pooled[2], pooled[3]))
            zs = [p - m for p in pooled]
            lse = jnp.log(sum(jnp.exp(z) for z in zs))
            for cl in range(4):
                out_ref[cl:cl + 1, pl.ds(gb * 128, 128)] = zs[cl] - lse

    return pl.pallas_call(
        body,
        out_shape=jax.ShapeDtypeStruct((4, G), _F32),
        scratch_shapes=[pltpu.VMEM((NP, 4), _F32)],
    )(acc, b3, batch2d)


def kernel(x, edge_index, batch, W1, a_src1, a_dst1, b1, W2, a_src2, a_dst2,
           b2, W3, a_src3, a_dst3, b3):
    loop = jnp.arange(N, dtype=_I32)
    padv = jnp.full((EP - EL,), N, _I32)
    srcf = jnp.concatenate([edge_index[0].astype(_I32), loop, padv])
    dstf = jnp.concatenate([edge_index[1].astype(_I32), loop, padv])
    batch2d = jnp.concatenate(
        [batch.astype(_I32), jnp.full((NP - N,), 1 << 30, _I32)])[:, None]
    xp = jnp.concatenate([x, jnp.zeros((NP - N, x.shape[1]), _F32)])

    src80 = srcf.reshape(EP // 80, 80)
    dst80 = dstf.reshape(EP // 80, 80)
    src128 = srcf.reshape(EP // 128, 128)
    dst128 = dstf.reshape(EP // 128, 128)

    hrow1, B1 = _tc_prep(xp, W1, a_src1, a_dst1)
    acc1 = _make_edge_kernel(3, 48, 80, 80)(src80, dst80, hrow1, B1)
    hrow2, B2 = _tc_mid(acc1, b1, W2, a_src2, a_dst2, 3, 45, 48, 3, 18, 32)
    acc2 = _make_edge_kernel(3, 32, 160, 80)(src80, dst80, hrow2, B2)
    hrow3, B3 = _tc_mid(acc2, b2, W3, a_src3, a_dst3, 3, 18, 32, 1, 4, 16)
    acc3 = _make_edge_kernel(1, 16, 512, 128)(src128, dst128, hrow3, B3)
    outT = _tc_pool(acc3, b3, batch2d)
    return outT.T
